# baseline (speedup 1.0000x reference)
import functools

import jax
import jax.numpy as jnp
from jax import lax
from jax.experimental import pallas as pl
from jax.experimental.pallas import tpu as pltpu
from jax.experimental.pallas import tpu_sc as plsc

NC, NS = 2, 16
NW = NC * NS
CHUNK = 128
BLK = 512


def _mesh():
    return plsc.VectorSubcoreMesh(core_axis_name="c", subcore_axis_name="s")


def _sc_hist(eidx, ones128, zeros128, npad, nch):
    rps = npad // NS
    w = ones128.shape[1]

    @functools.partial(
        pl.kernel, mesh=_mesh(),
        out_type=jax.ShapeDtypeStruct((NC, npad, w), jnp.float32),
        scratch_types=[pltpu.VMEM((2, CHUNK), jnp.int32),
                       pltpu.VMEM((2, CHUNK), jnp.int32),
                       pltpu.VMEM((CHUNK, w), jnp.float32),
                       pltpu.VMEM_SHARED((npad, w), jnp.float32),
                       pltpu.SemaphoreType.DMA,
                       pltpu.SemaphoreType.DMA],
    )
    def hist_kernel(eidx_hbm, ones_hbm, zeros_hbm, o_hbm,
                    idx0, idx1, onesv, acc_sh, isem0, isem1):
        c = lax.axis_index("c")
        s = lax.axis_index("s")
        wid = s * NC + c
        cb = wid * (nch + 2)
        pltpu.sync_copy(zeros_hbm.at[pl.ds(s * rps, rps)],
                        acc_sh.at[pl.ds(s * rps, rps)])
        pltpu.sync_copy(ones_hbm, onesv)
        pltpu.sync_copy(eidx_hbm.at[cb], idx0)
        plsc.subcore_barrier()
        pltpu.async_copy(eidx_hbm.at[cb + 1], idx1, isem1)

        @pl.loop(0, nch // 2)
        def _(i):
            g = 2 * i
            pltpu.sync_copy(onesv, acc_sh.at[idx0.at[1]], add=True)
            pltpu.make_async_copy(eidx_hbm.at[cb + g + 1], idx1, isem1).wait()
            pltpu.async_copy(eidx_hbm.at[cb + g + 2], idx0, isem0)
            pltpu.sync_copy(onesv, acc_sh.at[idx1.at[1]], add=True)
            pltpu.make_async_copy(eidx_hbm.at[cb + g + 2], idx0, isem0).wait()
            pltpu.async_copy(eidx_hbm.at[cb + g + 3], idx1, isem1)

        pltpu.make_async_copy(eidx_hbm.at[cb + nch + 1], idx1, isem1).wait()
        plsc.subcore_barrier()
        pltpu.sync_copy(acc_sh.at[pl.ds(s * rps, rps)],
                        o_hbm.at[c].at[pl.ds(s * rps, rps)])

    return hist_kernel(eidx, ones128, zeros128)


def _sc_conv(hp, src_pad, dst_pad, zeros128, npad, nch):
    rps = npad // NS
    h = hp.shape[1]

    @functools.partial(
        pl.kernel, mesh=_mesh(),
        out_type=jax.ShapeDtypeStruct((NC, npad, h), jnp.float32),
        scratch_types=[pltpu.VMEM((CHUNK,), jnp.int32),
                       pltpu.VMEM((CHUNK,), jnp.int32),
                       pltpu.VMEM((CHUNK, h), jnp.float32),
                       pltpu.VMEM_SHARED((npad, h), jnp.float32),
                       pltpu.SemaphoreType.DMA],
    )
    def conv_kernel(hp_hbm, src_hbm, dst_hbm, zeros_hbm, o_hbm,
                    srcv, dstv, rows, acc_sh, sem):
        c = lax.axis_index("c")
        s = lax.axis_index("s")
        wid = s * NC + c
        base = wid * (nch + 1) * CHUNK
        pltpu.sync_copy(zeros_hbm.at[pl.ds(s * rps, rps)],
                        acc_sh.at[pl.ds(s * rps, rps)])
        plsc.subcore_barrier()

        @pl.loop(0, nch)
        def _(ci):
            off = base + ci * CHUNK
            pltpu.sync_copy(src_hbm.at[pl.ds(off, CHUNK)], srcv)
            pltpu.sync_copy(dst_hbm.at[pl.ds(off, CHUNK)], dstv)
            pltpu.async_copy(hp_hbm.at[srcv], rows, sem).wait()
            pltpu.sync_copy(rows, acc_sh.at[dstv], add=True)

        plsc.subcore_barrier()
        pltpu.sync_copy(acc_sh.at[pl.ds(s * rps, rps)],
                        o_hbm.at[c].at[pl.ds(s * rps, rps)])

    return conv_kernel(hp, src_pad, dst_pad, zeros128)


def _embed_body(x_ref, w_ref, b_ref, o_ref):
    o_ref[...] = (jnp.dot(x_ref[...], w_ref[...],
                          preferred_element_type=jnp.float32) + b_ref[...])


def _embed(x, W, b):
    npad, d = x.shape
    h = W.shape[1]
    return pl.pallas_call(
        _embed_body,
        grid=(npad // BLK,),
        in_specs=[pl.BlockSpec((BLK, d), lambda i: (i, 0)),
                  pl.BlockSpec((d, h), lambda i: (0, 0)),
                  pl.BlockSpec((1, h), lambda i: (0, 0))],
        out_specs=pl.BlockSpec((BLK, h), lambda i: (i, 0)),
        out_shape=jax.ShapeDtypeStruct((npad, h), jnp.float32),
    )(x, W, b.reshape(1, h))


def _prescale_body(h_ref, w_ref, ha_ref, hb_ref, hp_ref, dinv_ref):
    deg = ha_ref[:, 0:1] + hb_ref[:, 0:1] + 1.0
    dinv = 1.0 / jnp.sqrt(deg)
    hw = jnp.dot(h_ref[...], w_ref[...], preferred_element_type=jnp.float32)
    hp_ref[...] = hw * dinv
    dinv_ref[...] = dinv


def _prescale(h0, W, ha, hb):
    npad, h = h0.shape
    return pl.pallas_call(
        _prescale_body,
        grid=(npad // BLK,),
        in_specs=[pl.BlockSpec((BLK, h), lambda i: (i, 0)),
                  pl.BlockSpec((h, h), lambda i: (0, 0)),
                  pl.BlockSpec((BLK, 128), lambda i: (i, 0)),
                  pl.BlockSpec((BLK, 128), lambda i: (i, 0))],
        out_specs=[pl.BlockSpec((BLK, h), lambda i: (i, 0)),
                   pl.BlockSpec((BLK, 1), lambda i: (i, 0))],
        out_shape=[jax.ShapeDtypeStruct((npad, h), jnp.float32),
                   jax.ShapeDtypeStruct((npad, 1), jnp.float32)],
    )(h0, W, ha, hb)


def _conv_next_body(aa_ref, ab_ref, hp_ref, dv_ref, b_ref, w_ref, o_ref):
    dv = dv_ref[...]
    s = (aa_ref[...] + ab_ref[...] + hp_ref[...]) * dv + b_ref[...]
    s = jnp.maximum(s, 0.0)
    o_ref[...] = jnp.dot(s, w_ref[...],
                         preferred_element_type=jnp.float32) * dv


def _conv_next(aa, ab, hp, dinv, b, Wn):
    npad, h = hp.shape
    return pl.pallas_call(
        _conv_next_body,
        grid=(npad // BLK,),
        in_specs=[pl.BlockSpec((BLK, h), lambda i: (i, 0)),
                  pl.BlockSpec((BLK, h), lambda i: (i, 0)),
                  pl.BlockSpec((BLK, h), lambda i: (i, 0)),
                  pl.BlockSpec((BLK, 1), lambda i: (i, 0)),
                  pl.BlockSpec((1, h), lambda i: (0, 0)),
                  pl.BlockSpec((h, h), lambda i: (0, 0))],
        out_specs=pl.BlockSpec((BLK, h), lambda i: (i, 0)),
        out_shape=jax.ShapeDtypeStruct((npad, h), jnp.float32),
    )(aa, ab, hp, dinv, b.reshape(1, h), Wn)


def _layer_norm(t, g, b):
    mu = jnp.mean(t, axis=-1, keepdims=True)
    var = jnp.mean((t - mu) ** 2, axis=-1, keepdims=True)
    return (t - mu) / jnp.sqrt(var + 1e-5) * g + b


def _head_body(aa_ref, ab_ref, hp_ref, dv_ref, bg_ref, wm1_ref, bm1_ref,
               g1_ref, be1_ref, wm2_ref, bm2_ref, g2_ref, be2_ref,
               wm3_ref, bm3_ref, o_ref):
    dv = dv_ref[...]
    t = (aa_ref[...] + ab_ref[...] + hp_ref[...]) * dv + bg_ref[...]
    t = jnp.maximum(t, 0.0)
    t = jnp.dot(t, wm1_ref[...], preferred_element_type=jnp.float32) + bm1_ref[...]
    t = jnp.maximum(_layer_norm(t, g1_ref[...], be1_ref[...]), 0.0)
    t = jnp.dot(t, wm2_ref[...], preferred_element_type=jnp.float32) + bm2_ref[...]
    t = jnp.maximum(_layer_norm(t, g2_ref[...], be2_ref[...]), 0.0)
    o_ref[...] = jnp.dot(t, wm3_ref[...],
                         preferred_element_type=jnp.float32) + bm3_ref[...]


def _head(aa, ab, hp, dinv, b_g2, W_m1, b_m1, g1, be1, W_m2, b_m2, g2, be2,
          W_m3, b_m3):
    npad, h = hp.shape
    row = lambda i: (i, 0)
    fixed = lambda i: (0, 0)
    return pl.pallas_call(
        _head_body,
        grid=(npad // BLK,),
        in_specs=[pl.BlockSpec((BLK, h), row),
                  pl.BlockSpec((BLK, h), row),
                  pl.BlockSpec((BLK, h), row),
                  pl.BlockSpec((BLK, 1), row),
                  pl.BlockSpec((1, h), fixed),
                  pl.BlockSpec((h, h), fixed),
                  pl.BlockSpec((1, h), fixed),
                  pl.BlockSpec((1, h), fixed),
                  pl.BlockSpec((1, h), fixed),
                  pl.BlockSpec((h, h), fixed),
                  pl.BlockSpec((1, h), fixed),
                  pl.BlockSpec((1, h), fixed),
                  pl.BlockSpec((1, h), fixed),
                  pl.BlockSpec((h, h), fixed),
                  pl.BlockSpec((1, h), fixed)],
        out_specs=pl.BlockSpec((BLK, h), row),
        out_shape=jax.ShapeDtypeStruct((npad, h), jnp.float32),
    )(aa, ab, hp, dinv, b_g2.reshape(1, h), W_m1, b_m1.reshape(1, h),
      g1.reshape(1, h), be1.reshape(1, h), W_m2, b_m2.reshape(1, h),
      g2.reshape(1, h), be2.reshape(1, h), W_m3, b_m3.reshape(1, h))


def kernel(x, adj, W_embed, b_embed, W_g1, b_g1, W_g2, b_g2,
           W_m1, b_m1, g1, be1, W_m2, b_m2, g2, be2, W_m3, b_m3):
    n, d = x.shape
    e = adj.shape[1]
    npad = -(-(n + 1) // BLK) * BLK
    step = NW * CHUNK * 2
    epad = -(-e // step) * step
    nch = epad // (NW * CHUNK)

    spare = npad - n
    pad_src = jnp.full((epad - e,), n, jnp.int32)
    pad_dst = n + (jnp.arange(epad - e, dtype=jnp.int32) % spare)
    src = jnp.concatenate([adj[0].astype(jnp.int32), pad_src])
    dst = jnp.concatenate([adj[1].astype(jnp.int32), pad_dst])
    chunk_pad_dst = jnp.broadcast_to(
        n + (jnp.arange(CHUNK, dtype=jnp.int32) % spare), (NW, 1, CHUNK))
    chunk_pad_src = jnp.full((NW, 1, CHUNK), n, jnp.int32)
    e3 = jnp.stack([src.reshape(NW, nch, CHUNK),
                    dst.reshape(NW, nch, CHUNK)], axis=2)
    epadchunks = jnp.stack(
        [chunk_pad_src, chunk_pad_dst], axis=2)
    eidx = jnp.concatenate(
        [e3, epadchunks, epadchunks],
        axis=1).reshape(NW * (nch + 2), 2, CHUNK)
    srcf = jnp.concatenate(
        [src.reshape(NW, nch, CHUNK), chunk_pad_src], axis=1).reshape(-1)
    dstf = jnp.concatenate(
        [dst.reshape(NW, nch, CHUNK), chunk_pad_dst], axis=1).reshape(-1)
    xp = jnp.concatenate([x, jnp.zeros((npad - n, d), jnp.float32)], axis=0)

    ones128 = jnp.ones((CHUNK, 128), jnp.float32)
    zeros128 = jnp.zeros((npad, W_g1.shape[1]), jnp.float32)

    hist = _sc_hist(eidx, ones128, zeros128, npad, nch)
    h0 = _embed(xp, W_embed, b_embed)
    h1p, dinv = _prescale(h0, W_g1, hist[0], hist[1])
    acc = _sc_conv(h1p, srcf, dstf, zeros128, npad, nch)
    h2p = _conv_next(acc[0], acc[1], h1p, dinv, b_g1, W_g2)
    acc2 = _sc_conv(h2p, srcf, dstf, zeros128, npad, nch)
    out = _head(acc2[0], acc2[1], h2p, dinv, b_g2, W_m1, b_m1, g1, be1,
                W_m2, b_m2, g2, be2, W_m3, b_m3)
    return out[:n]


# speedup vs baseline: 1.9547x; 1.9547x over previous
import functools

import jax
import jax.numpy as jnp
from jax import lax
from jax.experimental import pallas as pl
from jax.experimental.pallas import tpu as pltpu
from jax.experimental.pallas import tpu_sc as plsc

NC, NS = 2, 16
NW = NC * NS
CHUNK = 128
BLK = 512


def _mesh():
    return plsc.VectorSubcoreMesh(core_axis_name="c", subcore_axis_name="s")


def _sc_hist(eidx, ones128, zeros128, npad, nch):
    rps = npad // NS
    w = ones128.shape[1]

    @functools.partial(
        pl.kernel, mesh=_mesh(),
        out_type=jax.ShapeDtypeStruct((NC, npad, w), jnp.float32),
        scratch_types=[pltpu.VMEM((2, CHUNK), jnp.int32),
                       pltpu.VMEM((2, CHUNK), jnp.int32),
                       pltpu.VMEM((CHUNK, w), jnp.float32),
                       pltpu.VMEM_SHARED((npad, w), jnp.float32),
                       pltpu.SemaphoreType.DMA,
                       pltpu.SemaphoreType.DMA],
    )
    def hist_kernel(eidx_hbm, ones_hbm, zeros_hbm, o_hbm,
                    idx0, idx1, onesv, acc_sh, isem0, isem1):
        c = lax.axis_index("c")
        s = lax.axis_index("s")
        wid = s * NC + c
        cb = wid * (nch + 2)
        pltpu.sync_copy(zeros_hbm.at[pl.ds(s * rps, rps)],
                        acc_sh.at[pl.ds(s * rps, rps)])
        pltpu.sync_copy(ones_hbm, onesv)
        pltpu.sync_copy(eidx_hbm.at[cb], idx0)
        plsc.subcore_barrier()
        pltpu.async_copy(eidx_hbm.at[cb + 1], idx1, isem1)

        @pl.loop(0, nch // 2)
        def _(i):
            g = 2 * i
            pltpu.sync_copy(onesv, acc_sh.at[idx0.at[1]], add=True)
            pltpu.make_async_copy(eidx_hbm.at[cb + g + 1], idx1, isem1).wait()
            pltpu.async_copy(eidx_hbm.at[cb + g + 2], idx0, isem0)
            pltpu.sync_copy(onesv, acc_sh.at[idx1.at[1]], add=True)
            pltpu.make_async_copy(eidx_hbm.at[cb + g + 2], idx0, isem0).wait()
            pltpu.async_copy(eidx_hbm.at[cb + g + 3], idx1, isem1)

        pltpu.make_async_copy(eidx_hbm.at[cb + nch + 1], idx1, isem1).wait()
        plsc.subcore_barrier()
        pltpu.sync_copy(acc_sh.at[pl.ds(s * rps, rps)],
                        o_hbm.at[c].at[pl.ds(s * rps, rps)])

    return hist_kernel(eidx, ones128, zeros128)


def _sc_conv(hp, src_pad, dst_pad, zeros128, npad, nch):
    rps = npad // NS
    h = hp.shape[1]

    @functools.partial(
        pl.kernel, mesh=_mesh(),
        out_type=jax.ShapeDtypeStruct((NC, npad, h), jnp.float32),
        scratch_types=[pltpu.VMEM((CHUNK,), jnp.int32),
                       pltpu.VMEM((CHUNK,), jnp.int32),
                       pltpu.VMEM((CHUNK, h), jnp.float32),
                       pltpu.VMEM_SHARED((npad, h), jnp.float32),
                       pltpu.SemaphoreType.DMA],
    )
    def conv_kernel(hp_hbm, src_hbm, dst_hbm, zeros_hbm, o_hbm,
                    srcv, dstv, rows, acc_sh, sem):
        c = lax.axis_index("c")
        s = lax.axis_index("s")
        wid = s * NC + c
        base = wid * (nch + 1) * CHUNK
        pltpu.sync_copy(zeros_hbm.at[pl.ds(s * rps, rps)],
                        acc_sh.at[pl.ds(s * rps, rps)])
        plsc.subcore_barrier()

        @pl.loop(0, nch)
        def _(ci):
            off = base + ci * CHUNK
            pltpu.sync_copy(src_hbm.at[pl.ds(off, CHUNK)], srcv)
            pltpu.sync_copy(dst_hbm.at[pl.ds(off, CHUNK)], dstv)
            pltpu.async_copy(hp_hbm.at[srcv], rows, sem).wait()
            pltpu.sync_copy(rows, acc_sh.at[dstv], add=True)

        plsc.subcore_barrier()
        pltpu.sync_copy(acc_sh.at[pl.ds(s * rps, rps)],
                        o_hbm.at[c].at[pl.ds(s * rps, rps)])

    return conv_kernel(hp, src_pad, dst_pad, zeros128)


def _embed_body(x_ref, w_ref, b_ref, o_ref):
    o_ref[...] = (jnp.dot(x_ref[...], w_ref[...],
                          preferred_element_type=jnp.float32) + b_ref[...])


def _embed(x, W, b):
    npad, d = x.shape
    h = W.shape[1]
    return pl.pallas_call(
        _embed_body,
        grid=(npad // BLK,),
        in_specs=[pl.BlockSpec((BLK, d), lambda i: (i, 0)),
                  pl.BlockSpec((d, h), lambda i: (0, 0)),
                  pl.BlockSpec((1, h), lambda i: (0, 0))],
        out_specs=pl.BlockSpec((BLK, h), lambda i: (i, 0)),
        out_shape=jax.ShapeDtypeStruct((npad, h), jnp.float32),
    )(x, W, b.reshape(1, h))


def _prescale_body(h_ref, w_ref, ha_ref, hb_ref, hp_ref, dinv_ref):
    deg = ha_ref[:, 0:1] + hb_ref[:, 0:1] + 1.0
    dinv = 1.0 / jnp.sqrt(deg)
    hw = jnp.dot(h_ref[...], w_ref[...], preferred_element_type=jnp.float32)
    hp_ref[...] = hw * dinv
    dinv_ref[...] = dinv


def _prescale(h0, W, ha, hb):
    npad, h = h0.shape
    return pl.pallas_call(
        _prescale_body,
        grid=(npad // BLK,),
        in_specs=[pl.BlockSpec((BLK, h), lambda i: (i, 0)),
                  pl.BlockSpec((h, h), lambda i: (0, 0)),
                  pl.BlockSpec((BLK, 128), lambda i: (i, 0)),
                  pl.BlockSpec((BLK, 128), lambda i: (i, 0))],
        out_specs=[pl.BlockSpec((BLK, h), lambda i: (i, 0)),
                   pl.BlockSpec((BLK, 1), lambda i: (i, 0))],
        out_shape=[jax.ShapeDtypeStruct((npad, h), jnp.float32),
                   jax.ShapeDtypeStruct((npad, 1), jnp.float32)],
    )(h0, W, ha, hb)


def _conv_next_body(aa_ref, ab_ref, hp_ref, dv_ref, b_ref, w_ref, o_ref):
    dv = dv_ref[...]
    s = (aa_ref[...] + ab_ref[...] + hp_ref[...]) * dv + b_ref[...]
    s = jnp.maximum(s, 0.0)
    o_ref[...] = jnp.dot(s, w_ref[...],
                         preferred_element_type=jnp.float32) * dv


def _conv_next(aa, ab, hp, dinv, b, Wn):
    npad, h = hp.shape
    return pl.pallas_call(
        _conv_next_body,
        grid=(npad // BLK,),
        in_specs=[pl.BlockSpec((BLK, h), lambda i: (i, 0)),
                  pl.BlockSpec((BLK, h), lambda i: (i, 0)),
                  pl.BlockSpec((BLK, h), lambda i: (i, 0)),
                  pl.BlockSpec((BLK, 1), lambda i: (i, 0)),
                  pl.BlockSpec((1, h), lambda i: (0, 0)),
                  pl.BlockSpec((h, h), lambda i: (0, 0))],
        out_specs=pl.BlockSpec((BLK, h), lambda i: (i, 0)),
        out_shape=jax.ShapeDtypeStruct((npad, h), jnp.float32),
    )(aa, ab, hp, dinv, b.reshape(1, h), Wn)


def _layer_norm(t, g, b):
    mu = jnp.mean(t, axis=-1, keepdims=True)
    var = jnp.mean((t - mu) ** 2, axis=-1, keepdims=True)
    return (t - mu) / jnp.sqrt(var + 1e-5) * g + b


def _head_body(aa_ref, ab_ref, hp_ref, dv_ref, bg_ref, wm1_ref, bm1_ref,
               g1_ref, be1_ref, wm2_ref, bm2_ref, g2_ref, be2_ref,
               wm3_ref, bm3_ref, o_ref):
    dv = dv_ref[...]
    t = (aa_ref[...] + ab_ref[...] + hp_ref[...]) * dv + bg_ref[...]
    t = jnp.maximum(t, 0.0)
    t = jnp.dot(t, wm1_ref[...], preferred_element_type=jnp.float32) + bm1_ref[...]
    t = jnp.maximum(_layer_norm(t, g1_ref[...], be1_ref[...]), 0.0)
    t = jnp.dot(t, wm2_ref[...], preferred_element_type=jnp.float32) + bm2_ref[...]
    t = jnp.maximum(_layer_norm(t, g2_ref[...], be2_ref[...]), 0.0)
    o_ref[...] = jnp.dot(t, wm3_ref[...],
                         preferred_element_type=jnp.float32) + bm3_ref[...]


def _head(aa, ab, hp, dinv, b_g2, W_m1, b_m1, g1, be1, W_m2, b_m2, g2, be2,
          W_m3, b_m3):
    npad, h = hp.shape
    row = lambda i: (i, 0)
    fixed = lambda i: (0, 0)
    return pl.pallas_call(
        _head_body,
        grid=(npad // BLK,),
        in_specs=[pl.BlockSpec((BLK, h), row),
                  pl.BlockSpec((BLK, h), row),
                  pl.BlockSpec((BLK, h), row),
                  pl.BlockSpec((BLK, 1), row),
                  pl.BlockSpec((1, h), fixed),
                  pl.BlockSpec((h, h), fixed),
                  pl.BlockSpec((1, h), fixed),
                  pl.BlockSpec((1, h), fixed),
                  pl.BlockSpec((1, h), fixed),
                  pl.BlockSpec((h, h), fixed),
                  pl.BlockSpec((1, h), fixed),
                  pl.BlockSpec((1, h), fixed),
                  pl.BlockSpec((1, h), fixed),
                  pl.BlockSpec((h, h), fixed),
                  pl.BlockSpec((1, h), fixed)],
        out_specs=pl.BlockSpec((BLK, h), row),
        out_shape=jax.ShapeDtypeStruct((npad, h), jnp.float32),
    )(aa, ab, hp, dinv, b_g2.reshape(1, h), W_m1, b_m1.reshape(1, h),
      g1.reshape(1, h), be1.reshape(1, h), W_m2, b_m2.reshape(1, h),
      g2.reshape(1, h), be2.reshape(1, h), W_m3, b_m3.reshape(1, h))


def kernel(x, adj, W_embed, b_embed, W_g1, b_g1, W_g2, b_g2,
           W_m1, b_m1, g1, be1, W_m2, b_m2, g2, be2, W_m3, b_m3):
    n, d = x.shape
    e = adj.shape[1]
    npad = -(-(n + 1) // BLK) * BLK
    step = NW * CHUNK * 2
    epad = -(-e // step) * step
    nch = epad // (NW * CHUNK)

    spare = npad - n
    pad_src = n + (jnp.arange(epad - e, dtype=jnp.int32) % spare)
    pad_dst = n + ((jnp.arange(epad - e, dtype=jnp.int32) + 7) % spare)
    src = jnp.concatenate([adj[0].astype(jnp.int32), pad_src])
    dst = jnp.concatenate([adj[1].astype(jnp.int32), pad_dst])
    chunk_pad_dst = jnp.broadcast_to(
        n + (jnp.arange(CHUNK, dtype=jnp.int32) % spare), (NW, 1, CHUNK))
    chunk_pad_src = chunk_pad_dst
    e3 = jnp.stack([src.reshape(NW, nch, CHUNK),
                    dst.reshape(NW, nch, CHUNK)], axis=2)
    epadchunks = jnp.stack(
        [chunk_pad_src, chunk_pad_dst], axis=2)
    eidx = jnp.concatenate(
        [e3, epadchunks, epadchunks],
        axis=1).reshape(NW * (nch + 2), 2, CHUNK)
    srcf = jnp.concatenate(
        [src.reshape(NW, nch, CHUNK), chunk_pad_src], axis=1).reshape(-1)
    dstf = jnp.concatenate(
        [dst.reshape(NW, nch, CHUNK), chunk_pad_dst], axis=1).reshape(-1)
    xp = jnp.concatenate([x, jnp.zeros((npad - n, d), jnp.float32)], axis=0)

    ones128 = jnp.ones((CHUNK, 128), jnp.float32)
    zeros128 = jnp.zeros((npad, W_g1.shape[1]), jnp.float32)

    hist = _sc_hist(eidx, ones128, zeros128, npad, nch)
    h0 = _embed(xp, W_embed, b_embed)
    h1p, dinv = _prescale(h0, W_g1, hist[0], hist[1])
    acc = _sc_conv(h1p, srcf, dstf, zeros128, npad, nch)
    h2p = _conv_next(acc[0], acc[1], h1p, dinv, b_g1, W_g2)
    acc2 = _sc_conv(h2p, srcf, dstf, zeros128, npad, nch)
    out = _head(acc2[0], acc2[1], h2p, dinv, b_g2, W_m1, b_m1, g1, be1,
                W_m2, b_m2, g2, be2, W_m3, b_m3)
    return out[:n]


# speedup vs baseline: 2.0084x; 1.0275x over previous
import functools

import jax
import jax.numpy as jnp
from jax import lax
from jax.experimental import pallas as pl
from jax.experimental.pallas import tpu as pltpu
from jax.experimental.pallas import tpu_sc as plsc

NC, NS = 2, 16
NW = NC * NS
CHUNK = 128
BLK = 512


def _mesh():
    return plsc.VectorSubcoreMesh(core_axis_name="c", subcore_axis_name="s")


def _sc_hist(eidx, ones128, zeros128, npad, nch):
    rps = npad // NS
    w = ones128.shape[1]

    @functools.partial(
        pl.kernel, mesh=_mesh(),
        out_type=jax.ShapeDtypeStruct((NC, npad, w), jnp.float32),
        scratch_types=[pltpu.VMEM((2, CHUNK), jnp.int32),
                       pltpu.VMEM((2, CHUNK), jnp.int32),
                       pltpu.VMEM((CHUNK, w), jnp.float32),
                       pltpu.VMEM_SHARED((npad, w), jnp.float32),
                       pltpu.SemaphoreType.DMA,
                       pltpu.SemaphoreType.DMA],
    )
    def hist_kernel(eidx_hbm, ones_hbm, zeros_hbm, o_hbm,
                    idx0, idx1, onesv, acc_sh, isem0, isem1):
        c = lax.axis_index("c")
        s = lax.axis_index("s")
        wid = s * NC + c
        cb = wid * (nch + 2)
        pltpu.sync_copy(zeros_hbm.at[pl.ds(s * rps, rps)],
                        acc_sh.at[pl.ds(s * rps, rps)])
        pltpu.sync_copy(ones_hbm, onesv)
        pltpu.sync_copy(eidx_hbm.at[cb], idx0)
        plsc.subcore_barrier()
        pltpu.async_copy(eidx_hbm.at[cb + 1], idx1, isem1)

        @pl.loop(0, nch // 2)
        def _(i):
            g = 2 * i
            pltpu.sync_copy(onesv, acc_sh.at[idx0.at[1]], add=True)
            pltpu.make_async_copy(eidx_hbm.at[cb + g + 1], idx1, isem1).wait()
            pltpu.async_copy(eidx_hbm.at[cb + g + 2], idx0, isem0)
            pltpu.sync_copy(onesv, acc_sh.at[idx1.at[1]], add=True)
            pltpu.make_async_copy(eidx_hbm.at[cb + g + 2], idx0, isem0).wait()
            pltpu.async_copy(eidx_hbm.at[cb + g + 3], idx1, isem1)

        pltpu.make_async_copy(eidx_hbm.at[cb + nch + 1], idx1, isem1).wait()
        plsc.subcore_barrier()
        pltpu.sync_copy(acc_sh.at[pl.ds(s * rps, rps)],
                        o_hbm.at[c].at[pl.ds(s * rps, rps)])

    return hist_kernel(eidx, ones128, zeros128)


def _sc_conv(hp, src_pad, dst_pad, zeros128, npad, nch):
    rps = npad // NS
    h = hp.shape[1]

    @functools.partial(
        pl.kernel, mesh=_mesh(),
        out_type=jax.ShapeDtypeStruct((NC, npad, h), jnp.float32),
        scratch_types=[pltpu.VMEM((CHUNK,), jnp.int32),
                       pltpu.VMEM((CHUNK,), jnp.int32),
                       pltpu.VMEM((CHUNK, h), jnp.float32),
                       pltpu.VMEM_SHARED((npad, h), jnp.float32),
                       pltpu.SemaphoreType.DMA],
    )
    def conv_kernel(hp_hbm, src_hbm, dst_hbm, zeros_hbm, o_hbm,
                    srcv, dstv, rows, acc_sh, sem):
        c = lax.axis_index("c")
        s = lax.axis_index("s")
        wid = s * NC + c
        base = wid * (nch + 1) * CHUNK
        pltpu.sync_copy(zeros_hbm.at[pl.ds(s * rps, rps)],
                        acc_sh.at[pl.ds(s * rps, rps)])
        plsc.subcore_barrier()

        @pl.loop(0, nch)
        def _(ci):
            off = base + ci * CHUNK
            pltpu.sync_copy(src_hbm.at[pl.ds(off, CHUNK)], srcv)
            pltpu.sync_copy(dst_hbm.at[pl.ds(off, CHUNK)], dstv)
            pltpu.async_copy(hp_hbm.at[srcv], rows, sem).wait()
            pltpu.sync_copy(rows, acc_sh.at[dstv], add=True)

        plsc.subcore_barrier()
        pltpu.sync_copy(acc_sh.at[pl.ds(s * rps, rps)],
                        o_hbm.at[c].at[pl.ds(s * rps, rps)])

    return conv_kernel(hp, src_pad, dst_pad, zeros128)


def _prescale_body(x_ref, we_ref, be_ref, wg_ref, hist_ref, hp_ref, dinv_ref):
    deg = hist_ref[0, :, 0:1] + hist_ref[1, :, 0:1] + 1.0
    dinv = 1.0 / jnp.sqrt(deg)
    wc = jnp.dot(we_ref[...], wg_ref[...], preferred_element_type=jnp.float32)
    bc = jnp.dot(be_ref[...], wg_ref[...], preferred_element_type=jnp.float32)
    hw = jnp.dot(x_ref[...], wc, preferred_element_type=jnp.float32) + bc
    hp_ref[...] = hw * dinv
    dinv_ref[...] = dinv


def _prescale(x, We, be, Wg, hist):
    npad, d = x.shape
    h = Wg.shape[1]
    return pl.pallas_call(
        _prescale_body,
        grid=(npad // BLK,),
        in_specs=[pl.BlockSpec((BLK, d), lambda i: (i, 0)),
                  pl.BlockSpec((d, h), lambda i: (0, 0)),
                  pl.BlockSpec((1, h), lambda i: (0, 0)),
                  pl.BlockSpec((h, h), lambda i: (0, 0)),
                  pl.BlockSpec((NC, BLK, 128), lambda i: (0, i, 0))],
        out_specs=[pl.BlockSpec((BLK, h), lambda i: (i, 0)),
                   pl.BlockSpec((BLK, 1), lambda i: (i, 0))],
        out_shape=[jax.ShapeDtypeStruct((npad, h), jnp.float32),
                   jax.ShapeDtypeStruct((npad, 1), jnp.float32)],
    )(x, We, be.reshape(1, h), Wg, hist)


def _conv_next_body(acc_ref, hp_ref, dv_ref, b_ref, w_ref, o_ref):
    dv = dv_ref[...]
    s = (acc_ref[0] + acc_ref[1] + hp_ref[...]) * dv + b_ref[...]
    s = jnp.maximum(s, 0.0)
    o_ref[...] = jnp.dot(s, w_ref[...],
                         preferred_element_type=jnp.float32) * dv


def _conv_next(acc, hp, dinv, b, Wn):
    npad, h = hp.shape
    return pl.pallas_call(
        _conv_next_body,
        grid=(npad // BLK,),
        in_specs=[pl.BlockSpec((NC, BLK, h), lambda i: (0, i, 0)),
                  pl.BlockSpec((BLK, h), lambda i: (i, 0)),
                  pl.BlockSpec((BLK, 1), lambda i: (i, 0)),
                  pl.BlockSpec((1, h), lambda i: (0, 0)),
                  pl.BlockSpec((h, h), lambda i: (0, 0))],
        out_specs=pl.BlockSpec((BLK, h), lambda i: (i, 0)),
        out_shape=jax.ShapeDtypeStruct((npad, h), jnp.float32),
    )(acc, hp, dinv, b.reshape(1, h), Wn)


def _layer_norm(t, g, b):
    mu = jnp.mean(t, axis=-1, keepdims=True)
    var = jnp.mean((t - mu) ** 2, axis=-1, keepdims=True)
    return (t - mu) / jnp.sqrt(var + 1e-5) * g + b


def _head_body(acc_ref, hp_ref, dv_ref, bg_ref, wm1_ref, bm1_ref,
               g1_ref, be1_ref, wm2_ref, bm2_ref, g2_ref, be2_ref,
               wm3_ref, bm3_ref, o_ref):
    dv = dv_ref[...]
    t = (acc_ref[0] + acc_ref[1] + hp_ref[...]) * dv + bg_ref[...]
    t = jnp.maximum(t, 0.0)
    t = jnp.dot(t, wm1_ref[...], preferred_element_type=jnp.float32) + bm1_ref[...]
    t = jnp.maximum(_layer_norm(t, g1_ref[...], be1_ref[...]), 0.0)
    t = jnp.dot(t, wm2_ref[...], preferred_element_type=jnp.float32) + bm2_ref[...]
    t = jnp.maximum(_layer_norm(t, g2_ref[...], be2_ref[...]), 0.0)
    o_ref[...] = jnp.dot(t, wm3_ref[...],
                         preferred_element_type=jnp.float32) + bm3_ref[...]


def _head(acc, hp, dinv, b_g2, W_m1, b_m1, g1, be1, W_m2, b_m2, g2, be2,
          W_m3, b_m3):
    npad, h = hp.shape
    row = lambda i: (i, 0)
    fixed = lambda i: (0, 0)
    return pl.pallas_call(
        _head_body,
        grid=(npad // BLK,),
        in_specs=[pl.BlockSpec((NC, BLK, h), lambda i: (0, i, 0)),
                  pl.BlockSpec((BLK, h), row),
                  pl.BlockSpec((BLK, 1), row),
                  pl.BlockSpec((1, h), fixed),
                  pl.BlockSpec((h, h), fixed),
                  pl.BlockSpec((1, h), fixed),
                  pl.BlockSpec((1, h), fixed),
                  pl.BlockSpec((1, h), fixed),
                  pl.BlockSpec((h, h), fixed),
                  pl.BlockSpec((1, h), fixed),
                  pl.BlockSpec((1, h), fixed),
                  pl.BlockSpec((1, h), fixed),
                  pl.BlockSpec((h, h), fixed),
                  pl.BlockSpec((1, h), fixed)],
        out_specs=pl.BlockSpec((BLK, h), row),
        out_shape=jax.ShapeDtypeStruct((npad, h), jnp.float32),
    )(acc, hp, dinv, b_g2.reshape(1, h), W_m1, b_m1.reshape(1, h),
      g1.reshape(1, h), be1.reshape(1, h), W_m2, b_m2.reshape(1, h),
      g2.reshape(1, h), be2.reshape(1, h), W_m3, b_m3.reshape(1, h))


def kernel(x, adj, W_embed, b_embed, W_g1, b_g1, W_g2, b_g2,
           W_m1, b_m1, g1, be1, W_m2, b_m2, g2, be2, W_m3, b_m3):
    n, d = x.shape
    e = adj.shape[1]
    npad = -(-(n + 1) // BLK) * BLK
    step = NW * CHUNK * 2
    epad = -(-e // step) * step
    nch = epad // (NW * CHUNK)

    spare = npad - n
    pad_src = n + (jnp.arange(epad - e, dtype=jnp.int32) % spare)
    pad_dst = n + ((jnp.arange(epad - e, dtype=jnp.int32) + 7) % spare)
    src = jnp.concatenate([adj[0].astype(jnp.int32), pad_src])
    dst = jnp.concatenate([adj[1].astype(jnp.int32), pad_dst])
    chunk_pad_dst = jnp.broadcast_to(
        n + (jnp.arange(CHUNK, dtype=jnp.int32) % spare), (NW, 1, CHUNK))
    chunk_pad_src = chunk_pad_dst
    e3 = jnp.stack([src.reshape(NW, nch, CHUNK),
                    dst.reshape(NW, nch, CHUNK)], axis=2)
    epadchunks = jnp.stack(
        [chunk_pad_src, chunk_pad_dst], axis=2)
    eidx = jnp.concatenate(
        [e3, epadchunks, epadchunks],
        axis=1).reshape(NW * (nch + 2), 2, CHUNK)
    srcf = jnp.concatenate(
        [src.reshape(NW, nch, CHUNK), chunk_pad_src], axis=1).reshape(-1)
    dstf = jnp.concatenate(
        [dst.reshape(NW, nch, CHUNK), chunk_pad_dst], axis=1).reshape(-1)
    xp = jnp.concatenate([x, jnp.zeros((npad - n, d), jnp.float32)], axis=0)

    ones128 = jnp.ones((CHUNK, 128), jnp.float32)
    zeros128 = jnp.zeros((npad, W_g1.shape[1]), jnp.float32)

    hist = _sc_hist(eidx, ones128, zeros128, npad, nch)
    h1p, dinv = _prescale(xp, W_embed, b_embed, W_g1, hist)
    acc = _sc_conv(h1p, srcf, dstf, zeros128, npad, nch)
    h2p = _conv_next(acc, h1p, dinv, b_g1, W_g2)
    acc2 = _sc_conv(h2p, srcf, dstf, zeros128, npad, nch)
    out = _head(acc2, h2p, dinv, b_g2, W_m1, b_m1, g1, be1,
                W_m2, b_m2, g2, be2, W_m3, b_m3)
    return out[:n]


# speedup vs baseline: 2.0108x; 1.0012x over previous
import functools

import jax
import jax.numpy as jnp
from jax import lax
from jax.experimental import pallas as pl
from jax.experimental.pallas import tpu as pltpu
from jax.experimental.pallas import tpu_sc as plsc

NC, NS = 2, 16
NW = NC * NS
CHUNK = 128
BLK = 512


def _mesh():
    return plsc.VectorSubcoreMesh(core_axis_name="c", subcore_axis_name="s")


def _sc_hist(eidx, ones128, zeros128, npad, nch):
    rps = npad // NS
    w = ones128.shape[1]

    @functools.partial(
        pl.kernel, mesh=_mesh(),
        out_type=jax.ShapeDtypeStruct((NC, npad, w), jnp.float32),
        scratch_types=[pltpu.VMEM((2, CHUNK), jnp.int32),
                       pltpu.VMEM((2, CHUNK), jnp.int32),
                       pltpu.VMEM((CHUNK, w), jnp.float32),
                       pltpu.VMEM_SHARED((npad, w), jnp.float32),
                       pltpu.SemaphoreType.DMA,
                       pltpu.SemaphoreType.DMA],
    )
    def hist_kernel(eidx_hbm, ones_hbm, zeros_hbm, o_hbm,
                    idx0, idx1, onesv, acc_sh, isem0, isem1):
        c = lax.axis_index("c")
        s = lax.axis_index("s")
        wid = s * NC + c
        cb = wid * (nch + 2)
        pltpu.sync_copy(zeros_hbm.at[pl.ds(s * rps, rps)],
                        acc_sh.at[pl.ds(s * rps, rps)])
        pltpu.sync_copy(ones_hbm, onesv)
        pltpu.sync_copy(eidx_hbm.at[cb], idx0)
        plsc.subcore_barrier()
        pltpu.async_copy(eidx_hbm.at[cb + 1], idx1, isem1)

        @pl.loop(0, nch // 2)
        def _(i):
            g = 2 * i
            pltpu.sync_copy(onesv, acc_sh.at[idx0.at[1]], add=True)
            pltpu.make_async_copy(eidx_hbm.at[cb + g + 1], idx1, isem1).wait()
            pltpu.async_copy(eidx_hbm.at[cb + g + 2], idx0, isem0)
            pltpu.sync_copy(onesv, acc_sh.at[idx1.at[1]], add=True)
            pltpu.make_async_copy(eidx_hbm.at[cb + g + 2], idx0, isem0).wait()
            pltpu.async_copy(eidx_hbm.at[cb + g + 3], idx1, isem1)

        pltpu.make_async_copy(eidx_hbm.at[cb + nch + 1], idx1, isem1).wait()
        plsc.subcore_barrier()
        pltpu.sync_copy(acc_sh.at[pl.ds(s * rps, rps)],
                        o_hbm.at[c].at[pl.ds(s * rps, rps)])

    return hist_kernel(eidx, ones128, zeros128)


def _sc_conv(hp, src_pad, dst_pad, zeros128, npad, nch):
    rps = npad // NS
    h = hp.shape[1]

    @functools.partial(
        pl.kernel, mesh=_mesh(),
        out_type=jax.ShapeDtypeStruct((NC, npad, h), jnp.float32),
        scratch_types=[pltpu.VMEM((CHUNK,), jnp.int32),
                       pltpu.VMEM((CHUNK,), jnp.int32),
                       pltpu.VMEM((CHUNK, h), jnp.float32),
                       pltpu.VMEM_SHARED((npad, h), jnp.float32),
                       pltpu.SemaphoreType.DMA],
    )
    def conv_kernel(hp_hbm, src_hbm, dst_hbm, zeros_hbm, o_hbm,
                    srcv, dstv, rows, acc_sh, sem):
        c = lax.axis_index("c")
        s = lax.axis_index("s")
        wid = s * NC + c
        base = wid * (nch + 1) * CHUNK
        pltpu.sync_copy(zeros_hbm.at[pl.ds(s * rps, rps)],
                        acc_sh.at[pl.ds(s * rps, rps)])
        plsc.subcore_barrier()

        @pl.loop(0, nch)
        def _(ci):
            off = base + ci * CHUNK
            pltpu.sync_copy(src_hbm.at[pl.ds(off, CHUNK)], srcv)
            pltpu.sync_copy(dst_hbm.at[pl.ds(off, CHUNK)], dstv)
            pltpu.async_copy(hp_hbm.at[srcv], rows, sem).wait()
            pltpu.sync_copy(rows, acc_sh.at[dstv], add=True)

        plsc.subcore_barrier()
        pltpu.sync_copy(acc_sh.at[pl.ds(s * rps, rps)],
                        o_hbm.at[c].at[pl.ds(s * rps, rps)])

    return conv_kernel(hp, src_pad, dst_pad, zeros128)


def _prescale_body(x_ref, we_ref, be_ref, wg_ref, hist_ref, hp_ref, dinv_ref):
    deg = hist_ref[0, :, 0:1] + hist_ref[1, :, 0:1] + 1.0
    dinv = 1.0 / jnp.sqrt(deg)
    h0 = (jnp.dot(x_ref[...], we_ref[...],
                  preferred_element_type=jnp.float32) + be_ref[...])
    hw = jnp.dot(h0, wg_ref[...], preferred_element_type=jnp.float32)
    hp_ref[...] = hw * dinv
    dinv_ref[...] = dinv


def _prescale(x, We, be, Wg, hist):
    npad, d = x.shape
    h = Wg.shape[1]
    return pl.pallas_call(
        _prescale_body,
        grid=(npad // BLK,),
        in_specs=[pl.BlockSpec((BLK, d), lambda i: (i, 0)),
                  pl.BlockSpec((d, h), lambda i: (0, 0)),
                  pl.BlockSpec((1, h), lambda i: (0, 0)),
                  pl.BlockSpec((h, h), lambda i: (0, 0)),
                  pl.BlockSpec((NC, BLK, 128), lambda i: (0, i, 0))],
        out_specs=[pl.BlockSpec((BLK, h), lambda i: (i, 0)),
                   pl.BlockSpec((BLK, 1), lambda i: (i, 0))],
        out_shape=[jax.ShapeDtypeStruct((npad, h), jnp.float32),
                   jax.ShapeDtypeStruct((npad, 1), jnp.float32)],
    )(x, We, be.reshape(1, h), Wg, hist)


def _conv_next_body(acc_ref, hp_ref, dv_ref, b_ref, w_ref, o_ref):
    dv = dv_ref[...]
    s = (acc_ref[0] + acc_ref[1] + hp_ref[...]) * dv + b_ref[...]
    s = jnp.maximum(s, 0.0)
    o_ref[...] = jnp.dot(s, w_ref[...],
                         preferred_element_type=jnp.float32) * dv


def _conv_next(acc, hp, dinv, b, Wn):
    npad, h = hp.shape
    return pl.pallas_call(
        _conv_next_body,
        grid=(npad // BLK,),
        in_specs=[pl.BlockSpec((NC, BLK, h), lambda i: (0, i, 0)),
                  pl.BlockSpec((BLK, h), lambda i: (i, 0)),
                  pl.BlockSpec((BLK, 1), lambda i: (i, 0)),
                  pl.BlockSpec((1, h), lambda i: (0, 0)),
                  pl.BlockSpec((h, h), lambda i: (0, 0))],
        out_specs=pl.BlockSpec((BLK, h), lambda i: (i, 0)),
        out_shape=jax.ShapeDtypeStruct((npad, h), jnp.float32),
    )(acc, hp, dinv, b.reshape(1, h), Wn)


def _layer_norm(t, g, b):
    mu = jnp.mean(t, axis=-1, keepdims=True)
    var = jnp.mean((t - mu) ** 2, axis=-1, keepdims=True)
    return (t - mu) / jnp.sqrt(var + 1e-5) * g + b


def _head_body(acc_ref, hp_ref, dv_ref, bg_ref, wm1_ref, bm1_ref,
               g1_ref, be1_ref, wm2_ref, bm2_ref, g2_ref, be2_ref,
               wm3_ref, bm3_ref, o_ref):
    dv = dv_ref[...]
    t = (acc_ref[0] + acc_ref[1] + hp_ref[...]) * dv + bg_ref[...]
    t = jnp.maximum(t, 0.0)
    t = jnp.dot(t, wm1_ref[...], preferred_element_type=jnp.float32) + bm1_ref[...]
    t = jnp.maximum(_layer_norm(t, g1_ref[...], be1_ref[...]), 0.0)
    t = jnp.dot(t, wm2_ref[...], preferred_element_type=jnp.float32) + bm2_ref[...]
    t = jnp.maximum(_layer_norm(t, g2_ref[...], be2_ref[...]), 0.0)
    o_ref[...] = jnp.dot(t, wm3_ref[...],
                         preferred_element_type=jnp.float32) + bm3_ref[...]


def _head(acc, hp, dinv, b_g2, W_m1, b_m1, g1, be1, W_m2, b_m2, g2, be2,
          W_m3, b_m3):
    npad, h = hp.shape
    row = lambda i: (i, 0)
    fixed = lambda i: (0, 0)
    return pl.pallas_call(
        _head_body,
        grid=(npad // BLK,),
        in_specs=[pl.BlockSpec((NC, BLK, h), lambda i: (0, i, 0)),
                  pl.BlockSpec((BLK, h), row),
                  pl.BlockSpec((BLK, 1), row),
                  pl.BlockSpec((1, h), fixed),
                  pl.BlockSpec((h, h), fixed),
                  pl.BlockSpec((1, h), fixed),
                  pl.BlockSpec((1, h), fixed),
                  pl.BlockSpec((1, h), fixed),
                  pl.BlockSpec((h, h), fixed),
                  pl.BlockSpec((1, h), fixed),
                  pl.BlockSpec((1, h), fixed),
                  pl.BlockSpec((1, h), fixed),
                  pl.BlockSpec((h, h), fixed),
                  pl.BlockSpec((1, h), fixed)],
        out_specs=pl.BlockSpec((BLK, h), row),
        out_shape=jax.ShapeDtypeStruct((npad, h), jnp.float32),
    )(acc, hp, dinv, b_g2.reshape(1, h), W_m1, b_m1.reshape(1, h),
      g1.reshape(1, h), be1.reshape(1, h), W_m2, b_m2.reshape(1, h),
      g2.reshape(1, h), be2.reshape(1, h), W_m3, b_m3.reshape(1, h))


def kernel(x, adj, W_embed, b_embed, W_g1, b_g1, W_g2, b_g2,
           W_m1, b_m1, g1, be1, W_m2, b_m2, g2, be2, W_m3, b_m3):
    n, d = x.shape
    e = adj.shape[1]
    npad = -(-(n + 1) // BLK) * BLK
    step = NW * CHUNK * 2
    epad = -(-e // step) * step
    nch = epad // (NW * CHUNK)

    spare = npad - n
    pad_src = n + (jnp.arange(epad - e, dtype=jnp.int32) % spare)
    pad_dst = n + ((jnp.arange(epad - e, dtype=jnp.int32) + 7) % spare)
    src = jnp.concatenate([adj[0].astype(jnp.int32), pad_src])
    dst = jnp.concatenate([adj[1].astype(jnp.int32), pad_dst])
    chunk_pad_dst = jnp.broadcast_to(
        n + (jnp.arange(CHUNK, dtype=jnp.int32) % spare), (NW, 1, CHUNK))
    chunk_pad_src = chunk_pad_dst
    e3 = jnp.stack([src.reshape(NW, nch, CHUNK),
                    dst.reshape(NW, nch, CHUNK)], axis=2)
    epadchunks = jnp.stack(
        [chunk_pad_src, chunk_pad_dst], axis=2)
    eidx = jnp.concatenate(
        [e3, epadchunks, epadchunks],
        axis=1).reshape(NW * (nch + 2), 2, CHUNK)
    srcf = jnp.concatenate(
        [src.reshape(NW, nch, CHUNK), chunk_pad_src], axis=1).reshape(-1)
    dstf = jnp.concatenate(
        [dst.reshape(NW, nch, CHUNK), chunk_pad_dst], axis=1).reshape(-1)
    xp = jnp.concatenate([x, jnp.zeros((npad - n, d), jnp.float32)], axis=0)

    ones128 = jnp.ones((CHUNK, 128), jnp.float32)
    zeros128 = jnp.zeros((npad, W_g1.shape[1]), jnp.float32)

    hist = _sc_hist(eidx, ones128, zeros128, npad, nch)
    h1p, dinv = _prescale(xp, W_embed, b_embed, W_g1, hist)
    acc = _sc_conv(h1p, srcf, dstf, zeros128, npad, nch)
    h2p = _conv_next(acc, h1p, dinv, b_g1, W_g2)
    acc2 = _sc_conv(h2p, srcf, dstf, zeros128, npad, nch)
    out = _head(acc2, h2p, dinv, b_g2, W_m1, b_m1, g1, be1,
                W_m2, b_m2, g2, be2, W_m3, b_m3)
    return out[:n]
